# baseline (speedup 1.0000x reference)
import functools

import jax
import jax.numpy as jnp
from jax import lax
from jax.experimental import pallas as pl
from jax.experimental.pallas import tpu as pltpu
from jax.experimental.pallas import tpu_sc as plsc

B = 16384
H = 64
NW = 32
BPW = B // NW
ICHUNK = 128
NCHUNK = BPW // ICHUNK
L = 16


def _make_sc_kernel():
    mesh = plsc.VectorSubcoreMesh(core_axis_name="c", subcore_axis_name="s")

    @functools.partial(
        pl.kernel,
        mesh=mesh,
        out_type=jax.ShapeDtypeStruct((B,), jnp.float32),
        compiler_params=pltpu.CompilerParams(
            needs_layout_passes=False, use_tc_tiling_on_sc=False),
        scratch_types=[
            pltpu.VMEM((NCHUNK, ICHUNK), jnp.int32),
            pltpu.VMEM((NCHUNK, ICHUNK), jnp.int32),
            pltpu.VMEM((BPW, H), jnp.float32),
            pltpu.VMEM((BPW, H), jnp.float32),
            pltpu.VMEM((9 * L,), jnp.float32),
            pltpu.VMEM((BPW,), jnp.float32),
            pltpu.SemaphoreType.DMA,
        ],
    )
    def k(u_hbm, v_hbm, wc_hbm, tu_hbm, tv_hbm, out_hbm,
          idx_u, idx_v, rows_u, rows_v, wc, res, sem):
        wid = lax.axis_index("s") * 2 + lax.axis_index("c")
        base = wid * BPW

        pltpu.sync_copy(u_hbm.at[wid], idx_u)
        pltpu.sync_copy(v_hbm.at[wid], idx_v)
        pltpu.sync_copy(wc_hbm, wc)

        copies = []
        for j in range(NCHUNK):
            copies.append(pltpu.async_copy(
                tu_hbm.at[idx_u.at[j]], rows_u.at[pl.ds(j * ICHUNK, ICHUNK)],
                sem))
            copies.append(pltpu.async_copy(
                tv_hbm.at[idx_v.at[j]], rows_v.at[pl.ds(j * ICHUNK, ICHUNK)],
                sem))
        for cp in copies:
            cp.wait()

        w = [wc[pl.ds(kk * L, L)] for kk in range(8)]
        cvec = wc[pl.ds(8 * L, L)]
        lane = lax.iota(jnp.int32, L)
        m_last = lane == (L - 1)

        def dot_body(s, _):
            acc = w[0] * rows_u[s, pl.ds(0, L)]
            acc += w[1] * rows_u[s, pl.ds(L, L)]
            acc += w[2] * rows_u[s, pl.ds(2 * L, L)]
            acc += w[3] * rows_u[s, pl.ds(3 * L, L)]
            acc += w[4] * rows_v[s, pl.ds(0, L)]
            acc += w[5] * rows_v[s, pl.ds(L, L)]
            acc += w[6] * rows_v[s, pl.ds(2 * L, L)]
            acc += w[7] * rows_v[s, pl.ds(3 * L, L)]
            cum = plsc.cumsum(acc)
            plsc.store_scatter(res, [jnp.full((L,), s, jnp.int32)], cum,
                               mask=m_last)
            return 0

        lax.fori_loop(0, BPW, dot_body, 0)

        def sig_body(g, _):
            t = res[pl.ds(g * L, L)]
            res[pl.ds(g * L, L)] = 1.0 / (1.0 + jnp.exp(-(t + cvec)))
            return 0

        lax.fori_loop(0, BPW // L, sig_body, 0)

        pltpu.sync_copy(res, out_hbm.at[pl.ds(base, BPW)])

    return k


_sc_kernel = _make_sc_kernel()


def kernel(u, v, table_u, table_v, W1, gamma, beta, W2):
    scale = (gamma * jax.lax.rsqrt(jnp.float32(1.0 + 1e-5))) * W2[0]
    weff = scale @ W1
    c = jnp.dot(beta, W2[0])
    wc = jnp.concatenate([weff, jnp.full((L,), c, jnp.float32)])

    u3 = u.reshape(NW, NCHUNK, ICHUNK).astype(jnp.int32)
    v3 = v.reshape(NW, NCHUNK, ICHUNK).astype(jnp.int32)

    out = _sc_kernel(u3, v3, wc, table_u, table_v)
    return out.reshape(B, 1)


# speedup vs baseline: 6.2670x; 6.2670x over previous
import functools

import jax
import jax.numpy as jnp
from jax import lax
from jax.experimental import pallas as pl
from jax.experimental.pallas import tpu as pltpu
from jax.experimental.pallas import tpu_sc as plsc

B = 16384
H = 64
V = 1000000
NW = 32
BPW = B // NW
ICHUNK = 128
NCHUNK = BPW // ICHUNK
L = 16
BK = 16384
GRID = (V + BK - 1) // BK


def _tc_matvec_body(w8_ref, tu_ref, tv_ref, qu_ref, qv_ref):
    qu_ref[...] = jnp.dot(w8_ref[0:8, :], tu_ref[...],
                          preferred_element_type=jnp.float32)[0]
    qv_ref[...] = jnp.dot(w8_ref[8:16, :], tv_ref[...],
                          preferred_element_type=jnp.float32)[0]


_tc_matvec = pl.pallas_call(
    _tc_matvec_body,
    grid=(GRID,),
    in_specs=[
        pl.BlockSpec((16, H), lambda i: (0, 0)),
        pl.BlockSpec((H, BK), lambda i: (0, i)),
        pl.BlockSpec((H, BK), lambda i: (0, i)),
    ],
    out_specs=[
        pl.BlockSpec((BK,), lambda i: (i,)),
        pl.BlockSpec((BK,), lambda i: (i,)),
    ],
    out_shape=[
        jax.ShapeDtypeStruct((V,), jnp.float32),
        jax.ShapeDtypeStruct((V,), jnp.float32),
    ],
    compiler_params=pltpu.CompilerParams(
        dimension_semantics=("arbitrary",)),
)


def _make_sc_kernel():
    mesh = plsc.VectorSubcoreMesh(core_axis_name="c", subcore_axis_name="s")

    @functools.partial(
        pl.kernel,
        mesh=mesh,
        out_type=jax.ShapeDtypeStruct((B,), jnp.float32),
        compiler_params=pltpu.CompilerParams(
            needs_layout_passes=False, use_tc_tiling_on_sc=False),
        scratch_types=[
            pltpu.VMEM((NCHUNK, ICHUNK), jnp.int32),
            pltpu.VMEM((NCHUNK, ICHUNK), jnp.int32),
            pltpu.VMEM((NCHUNK, ICHUNK), jnp.float32),
            pltpu.VMEM((NCHUNK, ICHUNK), jnp.float32),
            pltpu.VMEM((L,), jnp.float32),
            pltpu.VMEM((BPW,), jnp.float32),
            pltpu.SemaphoreType.DMA,
        ],
    )
    def k(u_hbm, v_hbm, cvec_hbm, qu_hbm, qv_hbm, out_hbm,
          idx_u, idx_v, val_u, val_v, cv, res, sem):
        wid = lax.axis_index("s") * 2 + lax.axis_index("c")
        base = wid * BPW

        pltpu.sync_copy(u_hbm.at[wid], idx_u)
        pltpu.sync_copy(v_hbm.at[wid], idx_v)
        pltpu.sync_copy(cvec_hbm, cv)

        copies = []
        for j in range(NCHUNK):
            copies.append(pltpu.async_copy(
                qu_hbm.at[idx_u.at[j]], val_u.at[j], sem))
            copies.append(pltpu.async_copy(
                qv_hbm.at[idx_v.at[j]], val_v.at[j], sem))
        for cp in copies:
            cp.wait()

        cvec = cv[...]
        for j in range(NCHUNK):
            for g in range(ICHUNK // L):
                a = val_u[j, pl.ds(g * L, L)]
                b = val_v[j, pl.ds(g * L, L)]
                z = 1.0 / (1.0 + jnp.exp(-(a + b + cvec)))
                res[pl.ds(j * ICHUNK + g * L, L)] = z

        pltpu.sync_copy(res, out_hbm.at[pl.ds(base, BPW)])

    return k


_sc_kernel = _make_sc_kernel()


def kernel(u, v, table_u, table_v, W1, gamma, beta, W2):
    scale = (gamma * jax.lax.rsqrt(jnp.float32(1.0 + 1e-5))) * W2[0]
    weff = scale @ W1
    c = jnp.dot(beta, W2[0])
    w8 = jnp.zeros((16, H), jnp.float32)
    w8 = w8.at[0].set(weff[:H]).at[8].set(weff[H:])
    cvec = jnp.full((L,), c, jnp.float32)

    qu, qv = _tc_matvec(w8, table_u.T, table_v.T)

    u3 = u.reshape(NW, NCHUNK, ICHUNK).astype(jnp.int32)
    v3 = v.reshape(NW, NCHUNK, ICHUNK).astype(jnp.int32)
    out = _sc_kernel(u3, v3, cvec, qu, qv)
    return out.reshape(B, 1)
